# baseline (device time: 15734 ns/iter reference)
import jax
import jax.numpy as jnp
from jax import lax
from jax.experimental import pallas as pl
from jax.experimental.pallas import tpu as pltpu


def kernel(x):
    _, m, nh = x.shape
    hm = m // 2
    hn = nh // 2

    def body(x_ref, out_ref, xb, rs_recv, yraw_recv, draw_recv, q,
             send_sems, recv_sems):
        my_x = lax.axis_index("x")
        my_y = lax.axis_index("y")
        x_tgt = (1 - my_x, my_y)
        y_tgt = (my_x, 1 - my_y)
        d_tgt = (1 - my_x, 1 - my_y)

        barrier_sem = pltpu.get_barrier_semaphore()
        for tgt in (x_tgt, y_tgt, d_tgt):
            pl.semaphore_signal(barrier_sem, inc=1, device_id=tgt,
                                device_id_type=pl.DeviceIdType.MESH)
        pl.semaphore_wait(barrier_sem, 3)

        xb[...] = x_ref[0].astype(jnp.bfloat16)

        their_rows = pl.ds((1 - my_x) * hm, hm)
        own_rows = pl.ds(my_x * hm, hm)
        own_cols = pl.ds(my_y * nh, nh)

        def rs_piece(j):
            cols = pl.ds(j * hn, hn)
            return pltpu.make_async_remote_copy(
                src_ref=xb.at[their_rows, cols],
                dst_ref=rs_recv.at[:, cols],
                send_sem=send_sems.at[j], recv_sem=recv_sems.at[j],
                device_id=x_tgt, device_id_type=pl.DeviceIdType.MESH,
            )

        r_x0 = rs_piece(0)
        r_x1 = rs_piece(1)
        r_y = pltpu.make_async_remote_copy(
            src_ref=xb.at[their_rows],
            dst_ref=yraw_recv,
            send_sem=send_sems.at[2], recv_sem=recv_sems.at[2],
            device_id=y_tgt, device_id_type=pl.DeviceIdType.MESH,
        )
        r_d = pltpu.make_async_remote_copy(
            src_ref=xb.at[own_rows],
            dst_ref=draw_recv,
            send_sem=send_sems.at[3], recv_sem=recv_sems.at[3],
            device_id=d_tgt, device_id_type=pl.DeviceIdType.MESH,
        )
        r_x0.start()
        r_x1.start()
        r_y.start()
        r_d.start()

        def sum_piece(j, sem, tgt):
            cols = pl.ds(j * hn, hn)
            out_cols = pl.ds(my_y * nh + j * hn, hn)
            return pltpu.make_async_remote_copy(
                src_ref=q.at[:, cols],
                dst_ref=out_ref.at[own_rows, out_cols],
                send_sem=send_sems.at[sem], recv_sem=recv_sems.at[sem],
                device_id=tgt, device_id_type=pl.DeviceIdType.MESH,
            )

        sums = []
        for j, (r_xj, sx_sem, sy_sem) in enumerate(((r_x0, 4, 6),
                                                    (r_x1, 5, 7))):
            cols = pl.ds(j * hn, hn)
            r_xj.wait_recv()
            q[:, cols] = xb[own_rows, cols] + rs_recv[:, cols]
            s_x = sum_piece(j, sx_sem, x_tgt)
            s_y = sum_piece(j, sy_sem, y_tgt)
            s_x.start()
            s_y.start()
            sums += [s_x, s_y]

        out_ref[own_rows, own_cols] = q[...]

        r_y.wait_recv()
        r_d.wait_recv()
        out_ref[their_rows, pl.ds((1 - my_y) * nh, nh)] = (
            yraw_recv[...] + draw_recv[...]
        )

        for s in sums:
            s.wait()
        r_x0.wait_send()
        r_x1.wait_send()
        r_y.wait_send()
        r_d.wait_send()

    return pl.pallas_call(
        body,
        out_shape=jax.ShapeDtypeStruct((m, 2 * nh), jnp.bfloat16),
        in_specs=[pl.BlockSpec(memory_space=pltpu.VMEM)],
        out_specs=pl.BlockSpec(memory_space=pltpu.VMEM),
        scratch_shapes=[
            pltpu.VMEM((m, nh), jnp.bfloat16),
            pltpu.VMEM((hm, nh), jnp.bfloat16),
            pltpu.VMEM((hm, nh), jnp.bfloat16),
            pltpu.VMEM((hm, nh), jnp.bfloat16),
            pltpu.VMEM((hm, nh), jnp.bfloat16),
            pltpu.SemaphoreType.DMA((8,)),
            pltpu.SemaphoreType.DMA((8,)),
        ],
        compiler_params=pltpu.CompilerParams(collective_id=0),
    )(x)


# device time: 15557 ns/iter; 1.0114x vs baseline; 1.0114x over previous
import jax
import jax.numpy as jnp
from jax import lax
from jax.experimental import pallas as pl
from jax.experimental.pallas import tpu as pltpu

K = 4


def kernel(x):
    _, m, nh = x.shape
    cs = nh // K

    def body(x_ref, out_ref, xb, rs_recv, send_sems, recv_sems):
        my_x = lax.axis_index("x")
        my_y = lax.axis_index("y")
        x_tgt = (1 - my_x, my_y)
        y_tgt = (my_x, 1 - my_y)

        xb[...] = x_ref[0].astype(jnp.bfloat16)

        barrier_sem = pltpu.get_barrier_semaphore()
        for tgt in (x_tgt, y_tgt):
            pl.semaphore_signal(barrier_sem, inc=1, device_id=tgt,
                                device_id_type=pl.DeviceIdType.MESH)
        pl.semaphore_wait(barrier_sem, 2)

        raws = []
        for k in range(K):
            cols = pl.ds(k * cs, cs)
            r = pltpu.make_async_remote_copy(
                src_ref=xb.at[:, cols],
                dst_ref=rs_recv.at[:, cols],
                send_sem=send_sems.at[k], recv_sem=recv_sems.at[k],
                device_id=x_tgt, device_id_type=pl.DeviceIdType.MESH,
            )
            r.start()
            raws.append(r)

        sums = []
        for k in range(K):
            cols = pl.ds(k * cs, cs)
            out_cols = pl.ds(my_y * nh + k * cs, cs)
            raws[k].wait_recv()
            out_ref[:, out_cols] = xb[:, cols] + rs_recv[:, cols]
            s = pltpu.make_async_remote_copy(
                src_ref=out_ref.at[:, out_cols],
                dst_ref=out_ref.at[:, out_cols],
                send_sem=send_sems.at[K + k], recv_sem=recv_sems.at[K + k],
                device_id=y_tgt, device_id_type=pl.DeviceIdType.MESH,
            )
            s.start()
            sums.append(s)

        for s in sums:
            s.wait()
        for r in raws:
            r.wait_send()

    return pl.pallas_call(
        body,
        out_shape=jax.ShapeDtypeStruct((m, 2 * nh), jnp.bfloat16),
        in_specs=[pl.BlockSpec(memory_space=pltpu.VMEM)],
        out_specs=pl.BlockSpec(memory_space=pltpu.VMEM),
        scratch_shapes=[
            pltpu.VMEM((m, nh), jnp.bfloat16),
            pltpu.VMEM((m, nh), jnp.bfloat16),
            pltpu.SemaphoreType.DMA((2 * K,)),
            pltpu.SemaphoreType.DMA((2 * K,)),
        ],
        compiler_params=pltpu.CompilerParams(collective_id=0),
    )(x)


# device time: 15484 ns/iter; 1.0161x vs baseline; 1.0047x over previous
import jax
import jax.numpy as jnp
from jax import lax
from jax.experimental import pallas as pl
from jax.experimental.pallas import tpu as pltpu

K = 4


def kernel(x):
    _, m, nh = x.shape
    rs = m // K

    def body(x_ref, out_ref, xb, rs_recv, send_sems, recv_sems):
        my_x = lax.axis_index("x")
        my_y = lax.axis_index("y")
        x_tgt = (1 - my_x, my_y)
        y_tgt = (my_x, 1 - my_y)

        xb[...] = x_ref[0].astype(jnp.bfloat16)

        barrier_sem = pltpu.get_barrier_semaphore()
        for tgt in (x_tgt, y_tgt):
            pl.semaphore_signal(barrier_sem, inc=1, device_id=tgt,
                                device_id_type=pl.DeviceIdType.MESH)
        pl.semaphore_wait(barrier_sem, 2)

        raws = []
        for k in range(K):
            rows = pl.ds(k * rs, rs)
            r = pltpu.make_async_remote_copy(
                src_ref=xb.at[rows],
                dst_ref=rs_recv.at[rows],
                send_sem=send_sems.at[k], recv_sem=recv_sems.at[k],
                device_id=x_tgt, device_id_type=pl.DeviceIdType.MESH,
            )
            r.start()
            raws.append(r)

        own_cols = pl.ds(my_y * nh, nh)
        sums = []
        for k in range(K):
            rows = pl.ds(k * rs, rs)
            raws[k].wait_recv()
            out_ref[rows, own_cols] = xb[rows] + rs_recv[rows]
            s = pltpu.make_async_remote_copy(
                src_ref=out_ref.at[rows, own_cols],
                dst_ref=out_ref.at[rows, own_cols],
                send_sem=send_sems.at[K + k], recv_sem=recv_sems.at[K + k],
                device_id=y_tgt, device_id_type=pl.DeviceIdType.MESH,
            )
            s.start()
            sums.append(s)

        for s in sums:
            s.wait()
        for r in raws:
            r.wait_send()

    return pl.pallas_call(
        body,
        out_shape=jax.ShapeDtypeStruct((m, 2 * nh), jnp.bfloat16),
        in_specs=[pl.BlockSpec(memory_space=pltpu.VMEM)],
        out_specs=pl.BlockSpec(memory_space=pltpu.VMEM),
        scratch_shapes=[
            pltpu.VMEM((m, nh), jnp.bfloat16),
            pltpu.VMEM((m, nh), jnp.bfloat16),
            pltpu.SemaphoreType.DMA((2 * K,)),
            pltpu.SemaphoreType.DMA((2 * K,)),
        ],
        compiler_params=pltpu.CompilerParams(collective_id=0),
    )(x)
